# baseline (speedup 1.0000x reference)
import functools

import jax
import jax.numpy as jnp
from jax import lax
from jax.experimental import pallas as pl
from jax.experimental.pallas import tpu as pltpu
from jax.experimental.pallas import tpu_sc as plsc

_ROWS, _COLS = 64, 32768
_K = 64
_LANES = 16
_CHUNKS = _COLS // _LANES
_NBINS = 256
_ROWS_PER_TILE = 2
_U = 8
_ZW = 8192


def _tile_body(x_hbm, outf_hbm, row0_v, row1_v, cidx_v, zbuf_v,
               pidx_v, pval_v, h0, h1, h2, h3,
               sem_in0, sem_in1, sem_z, sem_s):
    hists = (h0, h1, h2, h3)
    cid = lax.axis_index("c")
    sid = lax.axis_index("s")
    wid = sid * 2 + cid
    r0 = wid * _ROWS_PER_TILE

    lane = lax.iota(jnp.int32, _LANES)
    ones_i = jnp.ones((_LANES,), jnp.int32)
    zeros_i = jnp.zeros((_LANES,), jnp.int32)
    zeros_f = jnp.zeros((_LANES,), jnp.float32)

    cp0 = pltpu.async_copy(x_hbm.at[r0], row0_v, sem_in0)
    cp1 = pltpu.async_copy(x_hbm.at[r0 + 1], row1_v, sem_in1)

    def zb(j, c):
        base = j * (_U * _LANES)
        for t in range(_U):
            zbuf_v[pl.ds(base + t * _LANES, _LANES)] = zeros_f
        return c

    lax.fori_loop(0, _ZW // (_U * _LANES), zb, jnp.int32(0))
    zcopies = []
    for rr in range(_ROWS_PER_TILE):
        for j in range(_COLS // _ZW):
            zcopies.append(pltpu.async_copy(
                zbuf_v,
                outf_hbm.at[pl.ds((r0 + rr) * _COLS + j * _ZW, _ZW)],
                sem_z,
            ))

    def bin_total(d):
        s = hists[0][pl.ds(d * _LANES, _LANES)]
        for h in hists[1:]:
            s = s + h[pl.ds(d * _LANES, _LANES)]
        return jnp.sum(s)

    def scan_bins(d0, kk):
        def cond(st):
            d, acc = st
            return acc + bin_total(d) < kk

        def body(st):
            d, acc = st
            return d - 1, acc + bin_total(d)

        return lax.while_loop(cond, body, (d0, jnp.int32(0)))

    def zero_hist():
        def zh(j, c):
            base = j * (_U * _LANES)
            for t in range(_U):
                for h in hists:
                    h[pl.ds(base + t * _LANES, _LANES)] = zeros_i
            return c

        lax.fori_loop(0, _NBINS // _U, zh, jnp.int32(0))

    def tree_max(ms):
        while len(ms) > 1:
            ms = [jnp.maximum(a, b) for a, b in zip(ms[::2], ms[1::2])]
        return ms[0]

    def process(row_v, row):
        zero_hist()

        def pA(i, umax):
            base = i * (_U * _LANES)
            xs = [row_v[pl.ds(base + t * _LANES, _LANES)]
                  for t in range(_U)]
            vs = [jnp.where(x > 0.0, x, zeros_f) for x in xs]
            us = [plsc.bitcast(v, jnp.int32) for v in vs]
            idxs = [lax.shift_right_logical(u, 24) * _LANES + lane
                    for u in us]
            for t in range(_U):
                plsc.addupdate_scatter(hists[t % 4], [idxs[t]], ones_i)
            return jnp.maximum(umax, tree_max(us))

        umax = lax.fori_loop(0, _CHUNKS // _U, pA, zeros_i)
        um = jnp.max(umax)
        d_sel, acc = scan_bins(
            lax.shift_right_logical(um, 24), jnp.int32(_K)
        )
        kk = jnp.int32(_K) - acc

        def pB(i, off):
            base = i * (_U * _LANES)
            os_ = [base + t * _LANES for t in range(_U)]
            xs = [row_v[pl.ds(o, _LANES)] for o in os_]
            vs = [jnp.where(x > 0.0, x, zeros_f) for x in xs]
            us = [plsc.bitcast(v, jnp.int32) for v in vs]
            ges = [lax.shift_right_logical(u, 24) >= d_sel for u in us]
            geis = [jnp.where(g, ones_i, zeros_i) for g in ges]
            css = [plsc.cumsum(g) for g in geis]
            pcs = [plsc.all_reduce_population_count(g) for g in ges]
            offs = [off]
            for t in range(_U):
                offs.append(offs[-1] + pcs[t])
            for t in range(_U):
                pos = offs[t] + css[t] - geis[t]
                plsc.store_scatter(
                    cidx_v, [pos], os_[t] + lane, mask=ges[t]
                )
            return offs[_U]

        offv = lax.fori_loop(0, _CHUNKS // _U, pB, zeros_i)
        ncand = jnp.max(offv)

        ncq = (ncand + _LANES - 1) // _LANES
        prefix = d_sel
        for p in range(1, 4):
            shift = 24 - 8 * p
            hs = shift + 8
            zero_hist()

            def pc(ci, umax, shift=shift, hs=hs, prefix=prefix,
                   ncand=ncand):
                cbase = ci * _LANES
                vm = (cbase + lane) < ncand
                cidx = cidx_v[pl.ds(cbase, _LANES)] & (_COLS - 1)
                xg = plsc.load_gather(row_v, [cidx], mask=vm)
                v = jnp.where(xg > 0.0, xg, zeros_f)
                u = plsc.bitcast(v, jnp.int32)
                cand = vm & (lax.shift_right_logical(u, hs) == prefix)
                dg = lax.shift_right_logical(u, shift) & 0xFF
                plsc.addupdate_scatter(
                    hists[0], [dg * _LANES + lane], ones_i, mask=cand
                )
                return jnp.maximum(umax, jnp.where(cand, u, zeros_i))

            umax = lax.fori_loop(0, ncq, pc, zeros_i)
            um = jnp.max(umax)
            d_sel2, acc = scan_bins(
                lax.shift_right_logical(um, shift) & 0xFF, kk
            )
            kk = kk - acc
            prefix = lax.shift_left(prefix, 8) | d_sel2


        def pr(ci, st, prefix=prefix, kk=kk, ncand=ncand, row=row):
            carry, wcnt = st
            cbase = ci * _LANES
            vm = (cbase + lane) < ncand
            cidx = cidx_v[pl.ds(cbase, _LANES)] & (_COLS - 1)
            xg = plsc.load_gather(row_v, [cidx], mask=vm)
            v = jnp.where(xg > 0.0, xg, zeros_f)
            u = plsc.bitcast(v, jnp.int32)
            gt = vm & (u > prefix)
            eq = vm & (u == prefix)
            cs = plsc.cumsum(jnp.where(eq, ones_i, zeros_i))
            keep = jnp.logical_or(gt, eq & ((cs + carry) <= kk))
            keepi = jnp.where(keep, ones_i, zeros_i)
            kpos = wcnt + plsc.cumsum(keepi) - keepi
            plsc.store_scatter(pidx_v, [kpos], row * _COLS + cidx,
                               mask=keep)
            plsc.store_scatter(pval_v, [kpos], v, mask=keep)
            return (carry + plsc.all_reduce_population_count(eq),
                    wcnt + plsc.all_reduce_population_count(keep))

        lax.fori_loop(0, ncq, pr, (zeros_i, zeros_i))

    cp0.wait()
    process(row0_v, r0)
    for c in zcopies:
        c.wait()
    pltpu.async_copy(pval_v, outf_hbm.at[pidx_v], sem_s).wait()
    cp1.wait()
    process(row1_v, r0 + 1)
    pltpu.async_copy(pval_v, outf_hbm.at[pidx_v], sem_s).wait()


@jax.jit
def _topk_sc(x):
    mesh = plsc.VectorSubcoreMesh(core_axis_name="c", subcore_axis_name="s")
    fn = pl.kernel(
        _tile_body,
        out_type=jax.ShapeDtypeStruct((_ROWS * _COLS,), jnp.float32),
        mesh=mesh,
        compiler_params=pltpu.CompilerParams(needs_layout_passes=False),
        scratch_types=[
            pltpu.VMEM((_COLS,), jnp.float32),
            pltpu.VMEM((_COLS,), jnp.float32),
            pltpu.VMEM((_COLS,), jnp.int32),
            pltpu.VMEM((_ZW,), jnp.float32),
            pltpu.VMEM((_K,), jnp.int32),
            pltpu.VMEM((_K,), jnp.float32),
            pltpu.VMEM((_NBINS * _LANES,), jnp.int32),
            pltpu.VMEM((_NBINS * _LANES,), jnp.int32),
            pltpu.VMEM((_NBINS * _LANES,), jnp.int32),
            pltpu.VMEM((_NBINS * _LANES,), jnp.int32),
            pltpu.SemaphoreType.DMA,
            pltpu.SemaphoreType.DMA,
            pltpu.SemaphoreType.DMA,
            pltpu.SemaphoreType.DMA,
        ],
    )
    return fn(x).reshape(_ROWS, _COLS)


def kernel(x):
    return _topk_sc(x)


# speedup vs baseline: 1.0498x; 1.0498x over previous
import functools

import jax
import jax.numpy as jnp
from jax import lax
from jax.experimental import pallas as pl
from jax.experimental.pallas import tpu as pltpu
from jax.experimental.pallas import tpu_sc as plsc

_ROWS, _COLS = 64, 32768
_K = 64
_LANES = 16
_CHUNKS = _COLS // _LANES
_NBINS = 256
_ROWS_PER_TILE = 2
_U = 8
_ZW = 8192


def _tile_body(x_hbm, outf_hbm, row0_v, row1_v, cidx_v, zbuf_v,
               pidx_v, pval_v, h0, h1, h2, h3,
               sem_in0, sem_in1, sem_z, sem_s):
    hists = (h0, h1, h2, h3)
    cid = lax.axis_index("c")
    sid = lax.axis_index("s")
    wid = sid * 2 + cid
    r0 = wid * _ROWS_PER_TILE

    lane = lax.iota(jnp.int32, _LANES)
    ones_i = jnp.ones((_LANES,), jnp.int32)
    zeros_i = jnp.zeros((_LANES,), jnp.int32)
    zeros_f = jnp.zeros((_LANES,), jnp.float32)

    cp0 = pltpu.async_copy(x_hbm.at[r0], row0_v, sem_in0)
    cp1 = pltpu.async_copy(x_hbm.at[r0 + 1], row1_v, sem_in1)

    def zb(j, c):
        base = j * (_U * _LANES)
        for t in range(_U):
            zbuf_v[pl.ds(base + t * _LANES, _LANES)] = zeros_f
        return c

    lax.fori_loop(0, _ZW // (_U * _LANES), zb, jnp.int32(0))
    zcopies = []
    for rr in range(_ROWS_PER_TILE):
        for j in range(_COLS // _ZW):
            zcopies.append(pltpu.async_copy(
                zbuf_v,
                outf_hbm.at[pl.ds((r0 + rr) * _COLS + j * _ZW, _ZW)],
                sem_z,
            ))

    def bin_total(d, nh):
        s = hists[0][pl.ds(d * _LANES, _LANES)]
        for h in hists[1:nh]:
            s = s + h[pl.ds(d * _LANES, _LANES)]
        return jnp.sum(s)

    def scan_bins(d0, kk, nh):
        def cond(st):
            d, acc = st
            return acc + bin_total(d, nh) < kk

        def body(st):
            d, acc = st
            return d - 1, acc + bin_total(d, nh)

        return lax.while_loop(cond, body, (d0, jnp.int32(0)))

    def zero_hist(nh):
        def zh(j, c):
            base = j * (_U * _LANES)
            for t in range(_U):
                for h in hists[:nh]:
                    h[pl.ds(base + t * _LANES, _LANES)] = zeros_i
            return c

        lax.fori_loop(0, _NBINS // _U, zh, jnp.int32(0))

    def tree_max(ms):
        while len(ms) > 1:
            ms = [jnp.maximum(a, b) for a, b in zip(ms[::2], ms[1::2])]
        return ms[0]

    def process(row_v, row):
        zero_hist(4)

        def pA(i, umax):
            base = i * (_U * _LANES)
            xs = [row_v[pl.ds(base + t * _LANES, _LANES)]
                  for t in range(_U)]
            us = [jnp.maximum(plsc.bitcast(x, jnp.int32), zeros_i)
                  for x in xs]
            idxs = [lax.shift_right_logical(u, 24) * _LANES + lane
                    for u in us]
            for t in range(_U):
                plsc.addupdate_scatter(hists[t % 4], [idxs[t]], ones_i)
            return jnp.maximum(umax, tree_max(us))

        umax = lax.fori_loop(0, _CHUNKS // _U, pA, zeros_i)
        um = jnp.max(umax)
        d_sel, acc = scan_bins(
            lax.shift_right_logical(um, 24), jnp.int32(_K), 4
        )
        kk = jnp.int32(_K) - acc

        def pB(i, off):
            base = i * (_U * _LANES)
            os_ = [base + t * _LANES for t in range(_U)]
            xs = [row_v[pl.ds(o, _LANES)] for o in os_]
            us = [jnp.maximum(plsc.bitcast(x, jnp.int32), zeros_i)
                  for x in xs]
            ges = [lax.shift_right_logical(u, 24) >= d_sel for u in us]
            pcs = [plsc.all_reduce_population_count(g) for g in ges]
            pss = [jnp.squeeze(lax.slice(p, (0,), (1,))) for p in pcs]
            offs = [off]
            for t in range(_U):
                offs.append(offs[-1] + pss[t])
            for t in range(_U):
                plsc.store_compressed(
                    cidx_v.at[pl.ds(offs[t], _LANES)],
                    os_[t] + lane, mask=ges[t]
                )
            return offs[_U]

        ncand = lax.fori_loop(0, _CHUNKS // _U, pB, jnp.int32(0))

        cu = 4
        ncq4 = (ncand + cu * _LANES - 1) // (cu * _LANES)
        prefix = d_sel
        for p in range(1, 4):
            shift = 24 - 8 * p
            hs = shift + 8
            zero_hist(cu)

            def pc(ci, umax, shift=shift, hs=hs, prefix=prefix,
                   ncand=ncand):
                cb = ci * (cu * _LANES)
                vms = [(cb + t * _LANES + lane) < ncand
                       for t in range(cu)]
                cidxs = [cidx_v[pl.ds(cb + t * _LANES, _LANES)]
                         & (_COLS - 1) for t in range(cu)]
                xgs = [plsc.load_gather(row_v, [cidxs[t]], mask=vms[t])
                       for t in range(cu)]
                us = [jnp.maximum(plsc.bitcast(x, jnp.int32), zeros_i)
                      for x in xgs]
                cands = [
                    vms[t]
                    & (lax.shift_right_logical(us[t], hs) == prefix)
                    for t in range(cu)
                ]
                dgs = [lax.shift_right_logical(u, shift) & 0xFF
                       for u in us]
                for t in range(cu):
                    plsc.addupdate_scatter(
                        hists[t], [dgs[t] * _LANES + lane], ones_i,
                        mask=cands[t],
                    )
                ms = [jnp.where(cands[t], us[t], zeros_i)
                      for t in range(cu)]
                return jnp.maximum(umax, tree_max(ms))

            umax = lax.fori_loop(0, ncq4, pc, zeros_i)
            um = jnp.max(umax)
            d_sel2, acc = scan_bins(
                lax.shift_right_logical(um, shift) & 0xFF, kk, cu
            )
            kk = kk - acc
            prefix = lax.shift_left(prefix, 8) | d_sel2


        def pr(ci, st, prefix=prefix, kk=kk, ncand=ncand, row=row):
            carry, wcnt = st
            cb = ci * (cu * _LANES)
            vms = [(cb + t * _LANES + lane) < ncand for t in range(cu)]
            cidxs = [cidx_v[pl.ds(cb + t * _LANES, _LANES)]
                     & (_COLS - 1) for t in range(cu)]
            xgs = [plsc.load_gather(row_v, [cidxs[t]], mask=vms[t])
                   for t in range(cu)]
            us = [jnp.maximum(plsc.bitcast(x, jnp.int32), zeros_i)
                  for x in xgs]
            gts = [vms[t] & (us[t] > prefix) for t in range(cu)]
            eqs = [vms[t] & (us[t] == prefix) for t in range(cu)]
            eqis = [jnp.where(e, ones_i, zeros_i) for e in eqs]
            css = [plsc.cumsum(e) for e in eqis]
            pce = [plsc.all_reduce_population_count(e) for e in eqs]
            carries = [carry]
            for t in range(cu):
                carries.append(carries[-1] + pce[t])
            keeps = [
                jnp.logical_or(
                    gts[t], eqs[t] & ((css[t] + carries[t]) <= kk)
                )
                for t in range(cu)
            ]
            keepis = [jnp.where(k, ones_i, zeros_i) for k in keeps]
            kcss = [plsc.cumsum(k) for k in keepis]
            pck = [plsc.all_reduce_population_count(k) for k in keeps]
            wcnts = [wcnt]
            for t in range(cu):
                wcnts.append(wcnts[-1] + pck[t])
            for t in range(cu):
                kpos = wcnts[t] + kcss[t] - keepis[t]
                plsc.store_scatter(
                    pidx_v, [kpos], row * _COLS + cidxs[t],
                    mask=keeps[t],
                )
                plsc.store_scatter(
                    pval_v, [kpos],
                    plsc.bitcast(us[t], jnp.float32), mask=keeps[t],
                )
            return (carries[cu], wcnts[cu])

        lax.fori_loop(0, ncq4, pr, (zeros_i, zeros_i))

    cp0.wait()
    process(row0_v, r0)
    for c in zcopies:
        c.wait()
    pltpu.async_copy(pval_v, outf_hbm.at[pidx_v], sem_s).wait()
    cp1.wait()
    process(row1_v, r0 + 1)
    pltpu.async_copy(pval_v, outf_hbm.at[pidx_v], sem_s).wait()


@jax.jit
def _topk_sc(x):
    mesh = plsc.VectorSubcoreMesh(core_axis_name="c", subcore_axis_name="s")
    fn = pl.kernel(
        _tile_body,
        out_type=jax.ShapeDtypeStruct((_ROWS * _COLS,), jnp.float32),
        mesh=mesh,
        compiler_params=pltpu.CompilerParams(needs_layout_passes=False),
        scratch_types=[
            pltpu.VMEM((_COLS,), jnp.float32),
            pltpu.VMEM((_COLS,), jnp.float32),
            pltpu.VMEM((_COLS + 64,), jnp.int32),
            pltpu.VMEM((_ZW,), jnp.float32),
            pltpu.VMEM((_K,), jnp.int32),
            pltpu.VMEM((_K,), jnp.float32),
            pltpu.VMEM((_NBINS * _LANES,), jnp.int32),
            pltpu.VMEM((_NBINS * _LANES,), jnp.int32),
            pltpu.VMEM((_NBINS * _LANES,), jnp.int32),
            pltpu.VMEM((_NBINS * _LANES,), jnp.int32),
            pltpu.SemaphoreType.DMA,
            pltpu.SemaphoreType.DMA,
            pltpu.SemaphoreType.DMA,
            pltpu.SemaphoreType.DMA,
        ],
    )
    return fn(x).reshape(_ROWS, _COLS)


def kernel(x):
    return _topk_sc(x)


# speedup vs baseline: 1.1004x; 1.0482x over previous
import functools

import jax
import jax.numpy as jnp
from jax import lax
from jax.experimental import pallas as pl
from jax.experimental.pallas import tpu as pltpu
from jax.experimental.pallas import tpu_sc as plsc

_ROWS, _COLS = 64, 32768
_K = 64
_LANES = 16
_CHUNKS = _COLS // _LANES
_NBINS = 256
_ROWS_PER_TILE = 2
_U = 8
_ZW = 8192


def _tile_body(x_hbm, outf_hbm, row0_v, row1_v, cidx_v, zbuf_v,
               pidx_v, pval_v, h0, h1, h2, h3,
               sem_in0, sem_in1, sem_z, sem_s):
    hists = (h0, h1, h2, h3)
    cid = lax.axis_index("c")
    sid = lax.axis_index("s")
    wid = sid * 2 + cid
    r0 = wid * _ROWS_PER_TILE

    lane = lax.iota(jnp.int32, _LANES)
    ones_i = jnp.ones((_LANES,), jnp.int32)
    zeros_i = jnp.zeros((_LANES,), jnp.int32)
    zeros_f = jnp.zeros((_LANES,), jnp.float32)

    cp0 = pltpu.async_copy(x_hbm.at[r0], row0_v, sem_in0)

    def zb(j, c):
        base = j * (_U * _LANES)
        for t in range(_U):
            zbuf_v[pl.ds(base + t * _LANES, _LANES)] = zeros_f
        return c

    lax.fori_loop(0, _ZW // (_U * _LANES), zb, jnp.int32(0))

    def bin_total(d, nh):
        s = hists[0][pl.ds(d * _LANES, _LANES)]
        for h in hists[1:nh]:
            s = s + h[pl.ds(d * _LANES, _LANES)]
        return jnp.sum(s)

    def scan_bins(d0, kk, nh):
        def cond(st):
            d, acc = st
            return acc + bin_total(d, nh) < kk

        def body(st):
            d, acc = st
            return d - 1, acc + bin_total(d, nh)

        return lax.while_loop(cond, body, (d0, jnp.int32(0)))

    def zero_hist(nh):
        def zh(j, c):
            base = j * (_U * _LANES)
            for t in range(_U):
                for h in hists[:nh]:
                    h[pl.ds(base + t * _LANES, _LANES)] = zeros_i
            return c

        lax.fori_loop(0, _NBINS // _U, zh, jnp.int32(0))

    def tree_max(ms):
        while len(ms) > 1:
            ms = [jnp.maximum(a, b) for a, b in zip(ms[::2], ms[1::2])]
        return ms[0]

    def process(row_v, row):
        zero_hist(4)

        def pA(i, umax):
            base = i * (_U * _LANES)
            xs = [row_v[pl.ds(base + t * _LANES, _LANES)]
                  for t in range(_U)]
            us = [jnp.maximum(plsc.bitcast(x, jnp.int32), zeros_i)
                  for x in xs]
            idxs = [lax.shift_right_logical(u, 24) * _LANES + lane
                    for u in us]
            for t in range(_U):
                plsc.addupdate_scatter(hists[t % 4], [idxs[t]], ones_i)
            return jnp.maximum(umax, tree_max(us))

        umax = lax.fori_loop(0, _CHUNKS // _U, pA, zeros_i)
        um = jnp.max(umax)
        d_sel, acc = scan_bins(
            lax.shift_right_logical(um, 24), jnp.int32(_K), 4
        )
        kk = jnp.int32(_K) - acc

        def pB(i, off):
            base = i * (_U * _LANES)
            os_ = [base + t * _LANES for t in range(_U)]
            xs = [row_v[pl.ds(o, _LANES)] for o in os_]
            us = [jnp.maximum(plsc.bitcast(x, jnp.int32), zeros_i)
                  for x in xs]
            ges = [lax.shift_right_logical(u, 24) >= d_sel for u in us]
            pcs = [plsc.all_reduce_population_count(g) for g in ges]
            pss = [jnp.squeeze(lax.slice(p, (0,), (1,))) for p in pcs]
            offs = [off]
            for t in range(_U):
                offs.append(offs[-1] + pss[t])
            for t in range(_U):
                plsc.store_compressed(
                    cidx_v.at[pl.ds(offs[t], _LANES)],
                    os_[t] + lane, mask=ges[t]
                )
            return offs[_U]

        ncand = lax.fori_loop(0, _CHUNKS // _U, pB, jnp.int32(0))

        cu = 4
        ncq4 = (ncand + cu * _LANES - 1) // (cu * _LANES)
        prefix = d_sel
        for p in range(1, 4):
            shift = 24 - 8 * p
            hs = shift + 8
            zero_hist(cu)

            def pc(ci, umax, shift=shift, hs=hs, prefix=prefix,
                   ncand=ncand):
                cb = ci * (cu * _LANES)
                vms = [(cb + t * _LANES + lane) < ncand
                       for t in range(cu)]
                cidxs = [cidx_v[pl.ds(cb + t * _LANES, _LANES)]
                         & (_COLS - 1) for t in range(cu)]
                xgs = [plsc.load_gather(row_v, [cidxs[t]], mask=vms[t])
                       for t in range(cu)]
                us = [jnp.maximum(plsc.bitcast(x, jnp.int32), zeros_i)
                      for x in xgs]
                cands = [
                    vms[t]
                    & (lax.shift_right_logical(us[t], hs) == prefix)
                    for t in range(cu)
                ]
                dgs = [lax.shift_right_logical(u, shift) & 0xFF
                       for u in us]
                for t in range(cu):
                    plsc.addupdate_scatter(
                        hists[t], [dgs[t] * _LANES + lane], ones_i,
                        mask=cands[t],
                    )
                ms = [jnp.where(cands[t], us[t], zeros_i)
                      for t in range(cu)]
                return jnp.maximum(umax, tree_max(ms))

            umax = lax.fori_loop(0, ncq4, pc, zeros_i)
            um = jnp.max(umax)
            d_sel2, acc = scan_bins(
                lax.shift_right_logical(um, shift) & 0xFF, kk, cu
            )
            kk = kk - acc
            prefix = lax.shift_left(prefix, 8) | d_sel2


        def pr(ci, st, prefix=prefix, kk=kk, ncand=ncand, row=row):
            carry, wcnt = st
            cb = ci * (cu * _LANES)
            vms = [(cb + t * _LANES + lane) < ncand for t in range(cu)]
            cidxs = [cidx_v[pl.ds(cb + t * _LANES, _LANES)]
                     & (_COLS - 1) for t in range(cu)]
            xgs = [plsc.load_gather(row_v, [cidxs[t]], mask=vms[t])
                   for t in range(cu)]
            us = [jnp.maximum(plsc.bitcast(x, jnp.int32), zeros_i)
                  for x in xgs]
            gts = [vms[t] & (us[t] > prefix) for t in range(cu)]
            eqs = [vms[t] & (us[t] == prefix) for t in range(cu)]
            eqis = [jnp.where(e, ones_i, zeros_i) for e in eqs]
            css = [plsc.cumsum(e) for e in eqis]
            pce = [plsc.all_reduce_population_count(e) for e in eqs]
            carries = [carry]
            for t in range(cu):
                carries.append(carries[-1] + pce[t])
            keeps = [
                jnp.logical_or(
                    gts[t], eqs[t] & ((css[t] + carries[t]) <= kk)
                )
                for t in range(cu)
            ]
            keepis = [jnp.where(k, ones_i, zeros_i) for k in keeps]
            kcss = [plsc.cumsum(k) for k in keepis]
            pck = [plsc.all_reduce_population_count(k) for k in keeps]
            wcnts = [wcnt]
            for t in range(cu):
                wcnts.append(wcnts[-1] + pck[t])
            for t in range(cu):
                kpos = wcnts[t] + kcss[t] - keepis[t]
                plsc.store_scatter(
                    pidx_v, [kpos], row * _COLS + cidxs[t],
                    mask=keeps[t],
                )
                plsc.store_scatter(
                    pval_v, [kpos],
                    plsc.bitcast(us[t], jnp.float32), mask=keeps[t],
                )
            return (carries[cu], wcnts[cu])

        lax.fori_loop(0, ncq4, pr, (zeros_i, zeros_i))

    cp0.wait()
    cp1 = pltpu.async_copy(x_hbm.at[r0 + 1], row1_v, sem_in1)
    zcopies = []
    for rr in range(_ROWS_PER_TILE):
        for j in range(_COLS // _ZW):
            zcopies.append(pltpu.async_copy(
                zbuf_v,
                outf_hbm.at[pl.ds((r0 + rr) * _COLS + j * _ZW, _ZW)],
                sem_z,
            ))
    process(row0_v, r0)
    for c in zcopies:
        c.wait()
    pltpu.async_copy(pval_v, outf_hbm.at[pidx_v], sem_s).wait()
    cp1.wait()
    process(row1_v, r0 + 1)
    pltpu.async_copy(pval_v, outf_hbm.at[pidx_v], sem_s).wait()


@jax.jit
def _topk_sc(x):
    mesh = plsc.VectorSubcoreMesh(core_axis_name="c", subcore_axis_name="s")
    fn = pl.kernel(
        _tile_body,
        out_type=jax.ShapeDtypeStruct((_ROWS * _COLS,), jnp.float32),
        mesh=mesh,
        compiler_params=pltpu.CompilerParams(needs_layout_passes=False),
        scratch_types=[
            pltpu.VMEM((_COLS,), jnp.float32),
            pltpu.VMEM((_COLS,), jnp.float32),
            pltpu.VMEM((_COLS + 64,), jnp.int32),
            pltpu.VMEM((_ZW,), jnp.float32),
            pltpu.VMEM((_K,), jnp.int32),
            pltpu.VMEM((_K,), jnp.float32),
            pltpu.VMEM((_NBINS * _LANES,), jnp.int32),
            pltpu.VMEM((_NBINS * _LANES,), jnp.int32),
            pltpu.VMEM((_NBINS * _LANES,), jnp.int32),
            pltpu.VMEM((_NBINS * _LANES,), jnp.int32),
            pltpu.SemaphoreType.DMA,
            pltpu.SemaphoreType.DMA,
            pltpu.SemaphoreType.DMA,
            pltpu.SemaphoreType.DMA,
        ],
    )
    return fn(x).reshape(_ROWS, _COLS)


def kernel(x):
    return _topk_sc(x)


# speedup vs baseline: 1.2133x; 1.1026x over previous
import functools

import jax
import jax.numpy as jnp
from jax import lax
from jax.experimental import pallas as pl
from jax.experimental.pallas import tpu as pltpu
from jax.experimental.pallas import tpu_sc as plsc

_ROWS, _COLS = 64, 32768
_K = 64
_LANES = 16
_CHUNKS = _COLS // _LANES
_NBINS = 256
_ROWS_PER_TILE = 2
_U = 16
_ZW = 8192


def _tile_body(x_hbm, outf_hbm, row0_v, row1_v, cidx_v, zbuf_v,
               pidx_v, pval_v, h0, h1, h2, h3,
               sem_in0, sem_in1, sem_z, sem_s):
    hists = (h0, h1, h2, h3)
    cid = lax.axis_index("c")
    sid = lax.axis_index("s")
    wid = sid * 2 + cid
    r0 = wid * _ROWS_PER_TILE

    lane = lax.iota(jnp.int32, _LANES)
    ones_i = jnp.ones((_LANES,), jnp.int32)
    zeros_i = jnp.zeros((_LANES,), jnp.int32)
    zeros_f = jnp.zeros((_LANES,), jnp.float32)

    cp0 = pltpu.async_copy(x_hbm.at[r0], row0_v, sem_in0)

    def zb(j, c):
        base = j * (_U * _LANES)
        for t in range(_U):
            zbuf_v[pl.ds(base + t * _LANES, _LANES)] = zeros_f
        return c

    lax.fori_loop(0, _ZW // (_U * _LANES), zb, jnp.int32(0))

    def bin_total(d, nh):
        s = hists[0][pl.ds(d * _LANES, _LANES)]
        for h in hists[1:nh]:
            s = s + h[pl.ds(d * _LANES, _LANES)]
        return jnp.sum(s)

    def scan_bins(d0, kk, nh):
        def cond(st):
            d, acc = st
            return acc + bin_total(d, nh) < kk

        def body(st):
            d, acc = st
            return d - 1, acc + bin_total(d, nh)

        return lax.while_loop(cond, body, (d0, jnp.int32(0)))

    def zero_hist(nh):
        def zh(j, c):
            base = j * (_U * _LANES)
            for t in range(_U):
                for h in hists[:nh]:
                    h[pl.ds(base + t * _LANES, _LANES)] = zeros_i
            return c

        lax.fori_loop(0, _NBINS // _U, zh, jnp.int32(0))

    def tree_max(ms):
        while len(ms) > 1:
            ms = [jnp.maximum(a, b) for a, b in zip(ms[::2], ms[1::2])]
        return ms[0]

    def process(row_v, row):
        zero_hist(4)

        def pA(i, umax):
            base = i * (_U * _LANES)
            xs = [row_v[pl.ds(base + t * _LANES, _LANES)]
                  for t in range(_U)]
            us = [jnp.maximum(plsc.bitcast(x, jnp.int32), zeros_i)
                  for x in xs]
            idxs = [lax.shift_right_logical(u, 24) * _LANES + lane
                    for u in us]
            for t in range(_U):
                plsc.addupdate_scatter(hists[t % 4], [idxs[t]], ones_i)
            return jnp.maximum(umax, tree_max(us))

        umax = lax.fori_loop(0, _CHUNKS // _U, pA, zeros_i)
        um = jnp.max(umax)
        d_sel, acc = scan_bins(
            lax.shift_right_logical(um, 24), jnp.int32(_K), 4
        )
        kk = jnp.int32(_K) - acc

        def pB(i, off):
            base = i * (_U * _LANES)
            os_ = [base + t * _LANES for t in range(_U)]
            xs = [row_v[pl.ds(o, _LANES)] for o in os_]
            us = [jnp.maximum(plsc.bitcast(x, jnp.int32), zeros_i)
                  for x in xs]
            ges = [lax.shift_right_logical(u, 24) >= d_sel for u in us]
            pcs = [plsc.all_reduce_population_count(g) for g in ges]
            pss = [jnp.squeeze(lax.slice(p, (0,), (1,))) for p in pcs]
            offs = [off]
            for t in range(_U):
                offs.append(offs[-1] + pss[t])
            for t in range(_U):
                plsc.store_compressed(
                    cidx_v.at[pl.ds(offs[t], _LANES)],
                    os_[t] + lane, mask=ges[t]
                )
            return offs[_U]

        ncand = lax.fori_loop(0, _CHUNKS // _U, pB, jnp.int32(0))

        cu = 4
        ncq4 = (ncand + cu * _LANES - 1) // (cu * _LANES)
        prefix = d_sel
        for p in range(1, 4):
            shift = 24 - 8 * p
            hs = shift + 8
            zero_hist(2)

            def pc(ci, umax, shift=shift, hs=hs, prefix=prefix,
                   ncand=ncand):
                cb = ci * (cu * _LANES)
                vms = [(cb + t * _LANES + lane) < ncand
                       for t in range(cu)]
                cidxs = [cidx_v[pl.ds(cb + t * _LANES, _LANES)]
                         & (_COLS - 1) for t in range(cu)]
                xgs = [plsc.load_gather(row_v, [cidxs[t]], mask=vms[t])
                       for t in range(cu)]
                us = [jnp.maximum(plsc.bitcast(x, jnp.int32), zeros_i)
                      for x in xgs]
                cands = [
                    vms[t]
                    & (lax.shift_right_logical(us[t], hs) == prefix)
                    for t in range(cu)
                ]
                dgs = [lax.shift_right_logical(u, shift) & 0xFF
                       for u in us]
                for t in range(cu):
                    plsc.addupdate_scatter(
                        hists[t % 2], [dgs[t] * _LANES + lane], ones_i,
                        mask=cands[t],
                    )
                ms = [jnp.where(cands[t], us[t], zeros_i)
                      for t in range(cu)]
                return jnp.maximum(umax, tree_max(ms))

            umax = lax.fori_loop(0, ncq4, pc, zeros_i)
            um = jnp.max(umax)
            d_sel2, acc = scan_bins(
                lax.shift_right_logical(um, shift) & 0xFF, kk, 2
            )
            kk = kk - acc
            prefix = lax.shift_left(prefix, 8) | d_sel2


        def pr(ci, st, prefix=prefix, kk=kk, ncand=ncand, row=row):
            carry, wcnt = st
            cb = ci * (cu * _LANES)
            vms = [(cb + t * _LANES + lane) < ncand for t in range(cu)]
            cidxs = [cidx_v[pl.ds(cb + t * _LANES, _LANES)]
                     & (_COLS - 1) for t in range(cu)]
            xgs = [plsc.load_gather(row_v, [cidxs[t]], mask=vms[t])
                   for t in range(cu)]
            us = [jnp.maximum(plsc.bitcast(x, jnp.int32), zeros_i)
                  for x in xgs]
            gts = [vms[t] & (us[t] > prefix) for t in range(cu)]
            eqs = [vms[t] & (us[t] == prefix) for t in range(cu)]
            eqis = [jnp.where(e, ones_i, zeros_i) for e in eqs]
            css = [plsc.cumsum(e) for e in eqis]
            pce = [plsc.all_reduce_population_count(e) for e in eqs]
            carries = [carry]
            for t in range(cu):
                carries.append(carries[-1] + pce[t])
            keeps = [
                jnp.logical_or(
                    gts[t], eqs[t] & ((css[t] + carries[t]) <= kk)
                )
                for t in range(cu)
            ]
            keepis = [jnp.where(k, ones_i, zeros_i) for k in keeps]
            kcss = [plsc.cumsum(k) for k in keepis]
            pck = [plsc.all_reduce_population_count(k) for k in keeps]
            wcnts = [wcnt]
            for t in range(cu):
                wcnts.append(wcnts[-1] + pck[t])
            for t in range(cu):
                kpos = wcnts[t] + kcss[t] - keepis[t]
                plsc.store_scatter(
                    pidx_v, [kpos], row * _COLS + cidxs[t],
                    mask=keeps[t],
                )
                plsc.store_scatter(
                    pval_v, [kpos],
                    plsc.bitcast(us[t], jnp.float32), mask=keeps[t],
                )
            return (carries[cu], wcnts[cu])

        lax.fori_loop(0, ncq4, pr, (zeros_i, zeros_i))

    cp0.wait()
    cp1 = pltpu.async_copy(x_hbm.at[r0 + 1], row1_v, sem_in1)
    zcopies = []
    for rr in range(_ROWS_PER_TILE):
        for j in range(_COLS // _ZW):
            zcopies.append(pltpu.async_copy(
                zbuf_v,
                outf_hbm.at[pl.ds((r0 + rr) * _COLS + j * _ZW, _ZW)],
                sem_z,
            ))
    process(row0_v, r0)
    for c in zcopies:
        c.wait()
    pltpu.async_copy(pval_v, outf_hbm.at[pidx_v], sem_s).wait()
    cp1.wait()
    process(row1_v, r0 + 1)
    pltpu.async_copy(pval_v, outf_hbm.at[pidx_v], sem_s).wait()


@jax.jit
def _topk_sc(x):
    mesh = plsc.VectorSubcoreMesh(core_axis_name="c", subcore_axis_name="s")
    fn = pl.kernel(
        _tile_body,
        out_type=jax.ShapeDtypeStruct((_ROWS * _COLS,), jnp.float32),
        mesh=mesh,
        compiler_params=pltpu.CompilerParams(needs_layout_passes=False),
        scratch_types=[
            pltpu.VMEM((_COLS,), jnp.float32),
            pltpu.VMEM((_COLS,), jnp.float32),
            pltpu.VMEM((_COLS + 64,), jnp.int32),
            pltpu.VMEM((_ZW,), jnp.float32),
            pltpu.VMEM((_K,), jnp.int32),
            pltpu.VMEM((_K,), jnp.float32),
            pltpu.VMEM((_NBINS * _LANES,), jnp.int32),
            pltpu.VMEM((_NBINS * _LANES,), jnp.int32),
            pltpu.VMEM((_NBINS * _LANES,), jnp.int32),
            pltpu.VMEM((_NBINS * _LANES,), jnp.int32),
            pltpu.SemaphoreType.DMA,
            pltpu.SemaphoreType.DMA,
            pltpu.SemaphoreType.DMA,
            pltpu.SemaphoreType.DMA,
        ],
    )
    return fn(x).reshape(_ROWS, _COLS)


def kernel(x):
    return _topk_sc(x)


# speedup vs baseline: 1.2963x; 1.0684x over previous
import functools

import jax
import jax.numpy as jnp
from jax import lax
from jax.experimental import pallas as pl
from jax.experimental.pallas import tpu as pltpu
from jax.experimental.pallas import tpu_sc as plsc

_ROWS, _COLS = 64, 32768
_K = 64
_LANES = 16
_CHUNKS = _COLS // _LANES
_NBINS = 256
_ROWS_PER_TILE = 2
_U = 16
_D_EST = 0x40
_SPEC_BITS = _D_EST << 24
_ZW = 8192


def _tile_body(x_hbm, outf_hbm, row0_v, row1_v, cidx_v, zbuf_v,
               pidx_v, pval_v, h0, h1, h2, h3,
               sem_in0, sem_in1, sem_z, sem_s):
    hists = (h0, h1, h2, h3)
    cid = lax.axis_index("c")
    sid = lax.axis_index("s")
    wid = sid * 2 + cid
    r0 = wid * _ROWS_PER_TILE

    lane = lax.iota(jnp.int32, _LANES)
    ones_i = jnp.ones((_LANES,), jnp.int32)
    zeros_i = jnp.zeros((_LANES,), jnp.int32)
    zeros_f = jnp.zeros((_LANES,), jnp.float32)

    cp0 = pltpu.async_copy(x_hbm.at[r0], row0_v, sem_in0)

    def zb(j, c):
        base = j * (_U * _LANES)
        for t in range(_U):
            zbuf_v[pl.ds(base + t * _LANES, _LANES)] = zeros_f
        return c

    lax.fori_loop(0, _ZW // (_U * _LANES), zb, jnp.int32(0))

    def bin_total(d, nh):
        s = hists[0][pl.ds(d * _LANES, _LANES)]
        for h in hists[1:nh]:
            s = s + h[pl.ds(d * _LANES, _LANES)]
        return jnp.sum(s)

    def scan_bins(d0, kk, nh):
        def cond(st):
            d, acc = st
            return acc + bin_total(d, nh) < kk

        def body(st):
            d, acc = st
            return d - 1, acc + bin_total(d, nh)

        return lax.while_loop(cond, body, (d0, jnp.int32(0)))

    def zero_hist(nh):
        def zh(j, c):
            base = j * (_U * _LANES)
            for t in range(_U):
                for h in hists[:nh]:
                    h[pl.ds(base + t * _LANES, _LANES)] = zeros_i
            return c

        lax.fori_loop(0, _NBINS // _U, zh, jnp.int32(0))

    def tree_max(ms):
        while len(ms) > 1:
            ms = [jnp.maximum(a, b) for a, b in zip(ms[::2], ms[1::2])]
        return ms[0]

    def process(row_v, row):
        zero_hist(4)

        def pAB(i, st):
            umax, off = st
            base = i * (_U * _LANES)
            os_ = [base + t * _LANES for t in range(_U)]
            xs = [row_v[pl.ds(o, _LANES)] for o in os_]
            us = [jnp.maximum(plsc.bitcast(x, jnp.int32), zeros_i)
                  for x in xs]
            idxs = [lax.shift_right_logical(u, 24) * _LANES + lane
                    for u in us]
            for t in range(_U):
                plsc.addupdate_scatter(hists[t % 4], [idxs[t]], ones_i)
            ges = [u >= jnp.int32(_SPEC_BITS) for u in us]
            pcs = [plsc.all_reduce_population_count(g) for g in ges]
            pss = [jnp.squeeze(lax.slice(p, (0,), (1,))) for p in pcs]
            offs = [off]
            for t in range(_U):
                offs.append(offs[-1] + pss[t])
            for t in range(_U):
                plsc.store_compressed(
                    cidx_v.at[pl.ds(offs[t], _LANES)],
                    os_[t] + lane, mask=ges[t]
                )
            return (jnp.maximum(umax, tree_max(us)), offs[_U])

        umax, nspec = lax.fori_loop(
            0, _CHUNKS // _U, pAB, (zeros_i, jnp.int32(0))
        )
        um = jnp.max(umax)
        d_sel, acc = scan_bins(
            lax.shift_right_logical(um, 24), jnp.int32(_K), 4
        )
        kk = jnp.int32(_K) - acc

        def pB(i, off):
            base = i * (_U * _LANES)
            os_ = [base + t * _LANES for t in range(_U)]
            xs = [row_v[pl.ds(o, _LANES)] for o in os_]
            us = [jnp.maximum(plsc.bitcast(x, jnp.int32), zeros_i)
                  for x in xs]
            ges = [lax.shift_right_logical(u, 24) >= d_sel for u in us]
            pcs = [plsc.all_reduce_population_count(g) for g in ges]
            pss = [jnp.squeeze(lax.slice(p, (0,), (1,))) for p in pcs]
            offs = [off]
            for t in range(_U):
                offs.append(offs[-1] + pss[t])
            for t in range(_U):
                plsc.store_compressed(
                    cidx_v.at[pl.ds(offs[t], _LANES)],
                    os_[t] + lane, mask=ges[t]
                )
            return offs[_U]

        spec_ok = d_sel >= jnp.int32(_D_EST)
        nfixq = jnp.where(spec_ok, 0, jnp.int32(_CHUNKS // _U))
        ncand_fb = lax.fori_loop(0, nfixq, pB, jnp.int32(0))
        ncand = jnp.where(spec_ok, nspec, ncand_fb)

        cu = 4
        ncq4 = (ncand + cu * _LANES - 1) // (cu * _LANES)
        prefix = d_sel
        for p in range(1, 4):
            shift = 24 - 8 * p
            hs = shift + 8
            zero_hist(2)

            def pc(ci, umax, shift=shift, hs=hs, prefix=prefix,
                   ncand=ncand):
                cb = ci * (cu * _LANES)
                vms = [(cb + t * _LANES + lane) < ncand
                       for t in range(cu)]
                cidxs = [cidx_v[pl.ds(cb + t * _LANES, _LANES)]
                         & (_COLS - 1) for t in range(cu)]
                xgs = [plsc.load_gather(row_v, [cidxs[t]], mask=vms[t])
                       for t in range(cu)]
                us = [jnp.maximum(plsc.bitcast(x, jnp.int32), zeros_i)
                      for x in xgs]
                cands = [
                    vms[t]
                    & (lax.shift_right_logical(us[t], hs) == prefix)
                    for t in range(cu)
                ]
                dgs = [lax.shift_right_logical(u, shift) & 0xFF
                       for u in us]
                for t in range(cu):
                    plsc.addupdate_scatter(
                        hists[t % 2], [dgs[t] * _LANES + lane], ones_i,
                        mask=cands[t],
                    )
                ms = [jnp.where(cands[t], us[t], zeros_i)
                      for t in range(cu)]
                return jnp.maximum(umax, tree_max(ms))

            umax = lax.fori_loop(0, ncq4, pc, zeros_i)
            um = jnp.max(umax)
            d_sel2, acc = scan_bins(
                lax.shift_right_logical(um, shift) & 0xFF, kk, 2
            )
            kk = kk - acc
            prefix = lax.shift_left(prefix, 8) | d_sel2


        def pr(ci, st, prefix=prefix, kk=kk, ncand=ncand, row=row):
            carry, wcnt = st
            cb = ci * (cu * _LANES)
            vms = [(cb + t * _LANES + lane) < ncand for t in range(cu)]
            cidxs = [cidx_v[pl.ds(cb + t * _LANES, _LANES)]
                     & (_COLS - 1) for t in range(cu)]
            xgs = [plsc.load_gather(row_v, [cidxs[t]], mask=vms[t])
                   for t in range(cu)]
            us = [jnp.maximum(plsc.bitcast(x, jnp.int32), zeros_i)
                  for x in xgs]
            gts = [vms[t] & (us[t] > prefix) for t in range(cu)]
            eqs = [vms[t] & (us[t] == prefix) for t in range(cu)]
            eqis = [jnp.where(e, ones_i, zeros_i) for e in eqs]
            css = [plsc.cumsum(e) for e in eqis]
            pce = [plsc.all_reduce_population_count(e) for e in eqs]
            carries = [carry]
            for t in range(cu):
                carries.append(carries[-1] + pce[t])
            keeps = [
                jnp.logical_or(
                    gts[t], eqs[t] & ((css[t] + carries[t]) <= kk)
                )
                for t in range(cu)
            ]
            keepis = [jnp.where(k, ones_i, zeros_i) for k in keeps]
            kcss = [plsc.cumsum(k) for k in keepis]
            pck = [plsc.all_reduce_population_count(k) for k in keeps]
            wcnts = [wcnt]
            for t in range(cu):
                wcnts.append(wcnts[-1] + pck[t])
            for t in range(cu):
                kpos = wcnts[t] + kcss[t] - keepis[t]
                plsc.store_scatter(
                    pidx_v, [kpos], row * _COLS + cidxs[t],
                    mask=keeps[t],
                )
                plsc.store_scatter(
                    pval_v, [kpos],
                    plsc.bitcast(us[t], jnp.float32), mask=keeps[t],
                )
            return (carries[cu], wcnts[cu])

        lax.fori_loop(0, ncq4, pr, (zeros_i, zeros_i))

    cp0.wait()
    cp1 = pltpu.async_copy(x_hbm.at[r0 + 1], row1_v, sem_in1)
    zcopies = []
    for rr in range(_ROWS_PER_TILE):
        for j in range(_COLS // _ZW):
            zcopies.append(pltpu.async_copy(
                zbuf_v,
                outf_hbm.at[pl.ds((r0 + rr) * _COLS + j * _ZW, _ZW)],
                sem_z,
            ))
    process(row0_v, r0)
    for c in zcopies:
        c.wait()
    pltpu.async_copy(pval_v, outf_hbm.at[pidx_v], sem_s).wait()
    cp1.wait()
    process(row1_v, r0 + 1)
    pltpu.async_copy(pval_v, outf_hbm.at[pidx_v], sem_s).wait()


@jax.jit
def _topk_sc(x):
    mesh = plsc.VectorSubcoreMesh(core_axis_name="c", subcore_axis_name="s")
    fn = pl.kernel(
        _tile_body,
        out_type=jax.ShapeDtypeStruct((_ROWS * _COLS,), jnp.float32),
        mesh=mesh,
        compiler_params=pltpu.CompilerParams(needs_layout_passes=False),
        scratch_types=[
            pltpu.VMEM((_COLS,), jnp.float32),
            pltpu.VMEM((_COLS,), jnp.float32),
            pltpu.VMEM((_COLS + 64,), jnp.int32),
            pltpu.VMEM((_ZW,), jnp.float32),
            pltpu.VMEM((_K,), jnp.int32),
            pltpu.VMEM((_K,), jnp.float32),
            pltpu.VMEM((_NBINS * _LANES,), jnp.int32),
            pltpu.VMEM((_NBINS * _LANES,), jnp.int32),
            pltpu.VMEM((_NBINS * _LANES,), jnp.int32),
            pltpu.VMEM((_NBINS * _LANES,), jnp.int32),
            pltpu.SemaphoreType.DMA,
            pltpu.SemaphoreType.DMA,
            pltpu.SemaphoreType.DMA,
            pltpu.SemaphoreType.DMA,
        ],
    )
    return fn(x).reshape(_ROWS, _COLS)


def kernel(x):
    return _topk_sc(x)


# speedup vs baseline: 1.3167x; 1.0157x over previous
import jax
import jax.numpy as jnp
from jax import lax
from jax.experimental import pallas as pl
from jax.experimental.pallas import tpu as pltpu
from jax.experimental.pallas import tpu_sc as plsc

_ROWS, _COLS = 64, 32768
_K = 64
_LANES = 16
_CHUNKS = _COLS // _LANES
_NBINS = 256
_ROWS_PER_TILE = 2
_U = 16
_D_EST = 0x40
_SPEC_BITS = _D_EST << 24
_ZW = 8192


def _tile_body(x_hbm, outf_hbm, row0_v, row1_v, cidx_v, zbuf_v,
               pidx_v, pval_v, h0, h1, h2, h3,
               sem_in0, sem_in1, sem_z, sem_s):
    hists = (h0, h1, h2, h3)
    cid = lax.axis_index("c")
    sid = lax.axis_index("s")
    wid = sid * 2 + cid
    r0 = wid * _ROWS_PER_TILE

    lane = lax.iota(jnp.int32, _LANES)
    ones_i = jnp.ones((_LANES,), jnp.int32)
    zeros_i = jnp.zeros((_LANES,), jnp.int32)
    zeros_f = jnp.zeros((_LANES,), jnp.float32)

    cp0 = pltpu.async_copy(x_hbm.at[r0], row0_v, sem_in0)

    def zb(j, c):
        base = j * (_U * _LANES)
        for t in range(_U):
            zbuf_v[pl.ds(base + t * _LANES, _LANES)] = zeros_f
        return c

    lax.fori_loop(0, _ZW // (_U * _LANES), zb, jnp.int32(0))

    def bin_total(d, nh):
        s = hists[0][pl.ds(d * _LANES, _LANES)]
        for h in hists[1:nh]:
            s = s + h[pl.ds(d * _LANES, _LANES)]
        return jnp.sum(s)

    def scan_bins(d0, kk, nh):
        def cond(st):
            d, acc = st
            return acc + bin_total(d, nh) < kk

        def body(st):
            d, acc = st
            return d - 1, acc + bin_total(d, nh)

        return lax.while_loop(cond, body, (d0, jnp.int32(0)))

    def zero_hist(nh):
        def zh(j, c):
            base = j * (_U * _LANES)
            for t in range(_U):
                for h in hists[:nh]:
                    h[pl.ds(base + t * _LANES, _LANES)] = zeros_i
            return c

        lax.fori_loop(0, _NBINS // _U, zh, jnp.int32(0))

    def tree_max(ms):
        while len(ms) > 1:
            ms = [jnp.maximum(a, b) for a, b in zip(ms[::2], ms[1::2])]
        return ms[0]

    def process(row_v, row):
        zero_hist(2)

        def pAB(i, st):
            umax, off = st
            base = i * (_U * _LANES)
            os_ = [base + t * _LANES for t in range(_U)]
            xs = [row_v[pl.ds(o, _LANES)] for o in os_]
            us = [jnp.maximum(plsc.bitcast(x, jnp.int32), zeros_i)
                  for x in xs]
            idxs = [lax.shift_right_logical(u, 24) * _LANES + lane
                    for u in us]
            for t in range(_U):
                plsc.addupdate_scatter(hists[t % 2], [idxs[t]], ones_i)
            ges = [u >= jnp.int32(_SPEC_BITS) for u in us]
            pcs = [plsc.all_reduce_population_count(g) for g in ges]
            pss = [jnp.squeeze(lax.slice(p, (0,), (1,))) for p in pcs]
            offs = [off]
            for t in range(_U):
                offs.append(offs[-1] + pss[t])
            for t in range(_U):
                plsc.store_compressed(
                    cidx_v.at[pl.ds(offs[t], _LANES)],
                    os_[t] + lane, mask=ges[t]
                )
            return (jnp.maximum(umax, tree_max(us)), offs[_U])

        umax, nspec = lax.fori_loop(
            0, _CHUNKS // _U, pAB, (zeros_i, jnp.int32(0))
        )
        um = jnp.max(umax)
        d_sel, acc = scan_bins(
            lax.shift_right_logical(um, 24), jnp.int32(_K), 2
        )
        kk = jnp.int32(_K) - acc

        def pB(i, off):
            base = i * (_U * _LANES)
            os_ = [base + t * _LANES for t in range(_U)]
            xs = [row_v[pl.ds(o, _LANES)] for o in os_]
            us = [jnp.maximum(plsc.bitcast(x, jnp.int32), zeros_i)
                  for x in xs]
            ges = [lax.shift_right_logical(u, 24) >= d_sel for u in us]
            pcs = [plsc.all_reduce_population_count(g) for g in ges]
            pss = [jnp.squeeze(lax.slice(p, (0,), (1,))) for p in pcs]
            offs = [off]
            for t in range(_U):
                offs.append(offs[-1] + pss[t])
            for t in range(_U):
                plsc.store_compressed(
                    cidx_v.at[pl.ds(offs[t], _LANES)],
                    os_[t] + lane, mask=ges[t]
                )
            return offs[_U]

        spec_ok = d_sel >= jnp.int32(_D_EST)
        nfixq = jnp.where(spec_ok, 0, jnp.int32(_CHUNKS // _U))
        ncand_fb = lax.fori_loop(0, nfixq, pB, jnp.int32(0))
        ncand = jnp.where(spec_ok, nspec, ncand_fb)

        cu = 4
        ncq4 = (ncand + cu * _LANES - 1) // (cu * _LANES)
        prefix = d_sel
        for p in range(1, 4):
            shift = 24 - 8 * p
            hs = shift + 8
            zero_hist(2)

            def pc(ci, umax, shift=shift, hs=hs, prefix=prefix,
                   ncand=ncand):
                cb = ci * (cu * _LANES)
                vms = [(cb + t * _LANES + lane) < ncand
                       for t in range(cu)]
                cidxs = [cidx_v[pl.ds(cb + t * _LANES, _LANES)]
                         & (_COLS - 1) for t in range(cu)]
                xgs = [plsc.load_gather(row_v, [cidxs[t]], mask=vms[t])
                       for t in range(cu)]
                us = [jnp.maximum(plsc.bitcast(x, jnp.int32), zeros_i)
                      for x in xgs]
                cands = [
                    vms[t]
                    & (lax.shift_right_logical(us[t], hs) == prefix)
                    for t in range(cu)
                ]
                dgs = [lax.shift_right_logical(u, shift) & 0xFF
                       for u in us]
                for t in range(cu):
                    plsc.addupdate_scatter(
                        hists[t % 2], [dgs[t] * _LANES + lane], ones_i,
                        mask=cands[t],
                    )
                ms = [jnp.where(cands[t], us[t], zeros_i)
                      for t in range(cu)]
                return jnp.maximum(umax, tree_max(ms))

            umax = lax.fori_loop(0, ncq4, pc, zeros_i)
            um = jnp.max(umax)
            d_sel2, acc = scan_bins(
                lax.shift_right_logical(um, shift) & 0xFF, kk, 2
            )
            kk = kk - acc
            prefix = lax.shift_left(prefix, 8) | d_sel2


        def pr(ci, st, prefix=prefix, kk=kk, ncand=ncand, row=row):
            carry, wcnt = st
            cb = ci * (cu * _LANES)
            vms = [(cb + t * _LANES + lane) < ncand for t in range(cu)]
            cidxs = [cidx_v[pl.ds(cb + t * _LANES, _LANES)]
                     & (_COLS - 1) for t in range(cu)]
            xgs = [plsc.load_gather(row_v, [cidxs[t]], mask=vms[t])
                   for t in range(cu)]
            us = [jnp.maximum(plsc.bitcast(x, jnp.int32), zeros_i)
                  for x in xgs]
            gts = [vms[t] & (us[t] > prefix) for t in range(cu)]
            eqs = [vms[t] & (us[t] == prefix) for t in range(cu)]
            eqis = [jnp.where(e, ones_i, zeros_i) for e in eqs]
            css = [plsc.cumsum(e) for e in eqis]
            pce = [plsc.all_reduce_population_count(e) for e in eqs]
            carries = [carry]
            for t in range(cu):
                carries.append(carries[-1] + pce[t])
            keeps = [
                jnp.logical_or(
                    gts[t], eqs[t] & ((css[t] + carries[t]) <= kk)
                )
                for t in range(cu)
            ]
            keepis = [jnp.where(k, ones_i, zeros_i) for k in keeps]
            kcss = [plsc.cumsum(k) for k in keepis]
            pck = [plsc.all_reduce_population_count(k) for k in keeps]
            wcnts = [wcnt]
            for t in range(cu):
                wcnts.append(wcnts[-1] + pck[t])
            for t in range(cu):
                kpos = wcnts[t] + kcss[t] - keepis[t]
                plsc.store_scatter(
                    pidx_v, [kpos], row * _COLS + cidxs[t],
                    mask=keeps[t],
                )
                plsc.store_scatter(
                    pval_v, [kpos],
                    plsc.bitcast(us[t], jnp.float32), mask=keeps[t],
                )
            return (carries[cu], wcnts[cu])

        lax.fori_loop(0, ncq4, pr, (zeros_i, zeros_i))

    cp0.wait()
    cp1 = pltpu.async_copy(x_hbm.at[r0 + 1], row1_v, sem_in1)
    zcopies = []
    for rr in range(_ROWS_PER_TILE):
        for j in range(_COLS // _ZW):
            zcopies.append(pltpu.async_copy(
                zbuf_v,
                outf_hbm.at[pl.ds((r0 + rr) * _COLS + j * _ZW, _ZW)],
                sem_z,
            ))
    process(row0_v, r0)
    for c in zcopies:
        c.wait()
    pltpu.async_copy(pval_v, outf_hbm.at[pidx_v], sem_s).wait()
    cp1.wait()
    process(row1_v, r0 + 1)
    pltpu.async_copy(pval_v, outf_hbm.at[pidx_v], sem_s).wait()


@jax.jit
def _topk_sc(x):
    mesh = plsc.VectorSubcoreMesh(core_axis_name="c", subcore_axis_name="s")
    fn = pl.kernel(
        _tile_body,
        out_type=jax.ShapeDtypeStruct((_ROWS * _COLS,), jnp.float32),
        mesh=mesh,
        compiler_params=pltpu.CompilerParams(needs_layout_passes=False),
        scratch_types=[
            pltpu.VMEM((_COLS,), jnp.float32),
            pltpu.VMEM((_COLS,), jnp.float32),
            pltpu.VMEM((_COLS + 64,), jnp.int32),
            pltpu.VMEM((_ZW,), jnp.float32),
            pltpu.VMEM((_K,), jnp.int32),
            pltpu.VMEM((_K,), jnp.float32),
            pltpu.VMEM((_NBINS * _LANES,), jnp.int32),
            pltpu.VMEM((_NBINS * _LANES,), jnp.int32),
            pltpu.VMEM((_NBINS * _LANES,), jnp.int32),
            pltpu.VMEM((_NBINS * _LANES,), jnp.int32),
            pltpu.SemaphoreType.DMA,
            pltpu.SemaphoreType.DMA,
            pltpu.SemaphoreType.DMA,
            pltpu.SemaphoreType.DMA,
        ],
    )
    return fn(x).reshape(_ROWS, _COLS)


def kernel(x):
    return _topk_sc(x)


# speedup vs baseline: 1.3270x; 1.0079x over previous
import jax
import jax.numpy as jnp
from jax import lax
from jax.experimental import pallas as pl
from jax.experimental.pallas import tpu as pltpu
from jax.experimental.pallas import tpu_sc as plsc

_ROWS, _COLS = 64, 32768
_K = 64
_LANES = 16
_CHUNKS = _COLS // _LANES
_NBINS = 256
_ROWS_PER_TILE = 2
_U = 16
_D_EST = 0x40
_SPEC_BITS = _D_EST << 24
_ZW = 8192


def _tile_body(x_hbm, outf_hbm, row0_v, row1_v, cidx_v, zbuf_v,
               pidx_v, pval_v, h0, h1, h2, h3,
               sem_in0, sem_in1, sem_z, sem_s):
    hists = (h0, h1, h2, h3)
    cid = lax.axis_index("c")
    sid = lax.axis_index("s")
    wid = sid * 2 + cid
    r0 = wid * _ROWS_PER_TILE

    lane = lax.iota(jnp.int32, _LANES)
    ones_i = jnp.ones((_LANES,), jnp.int32)
    zeros_i = jnp.zeros((_LANES,), jnp.int32)
    zeros_f = jnp.zeros((_LANES,), jnp.float32)

    cp0 = pltpu.async_copy(x_hbm.at[r0], row0_v, sem_in0)

    def zb(j, c):
        base = j * (_U * _LANES)
        for t in range(_U):
            zbuf_v[pl.ds(base + t * _LANES, _LANES)] = zeros_f
        return c

    lax.fori_loop(0, _ZW // (_U * _LANES), zb, jnp.int32(0))

    def bin_total(d, nh):
        s = hists[0][pl.ds(d * _LANES, _LANES)]
        for h in hists[1:nh]:
            s = s + h[pl.ds(d * _LANES, _LANES)]
        return jnp.sum(s)

    def scan_bins(d0, kk, nh):
        def cond(st):
            d, acc = st
            return acc + bin_total(d, nh) < kk

        def body(st):
            d, acc = st
            return d - 1, acc + bin_total(d, nh)

        return lax.while_loop(cond, body, (d0, jnp.int32(0)))

    def zero_hist(nh):
        def zh(j, c):
            base = j * (_U * _LANES)
            for t in range(_U):
                for h in hists[:nh]:
                    h[pl.ds(base + t * _LANES, _LANES)] = zeros_i
            return c

        lax.fori_loop(0, _NBINS // _U, zh, jnp.int32(0))

    def tree_max(ms):
        while len(ms) > 1:
            ms = [jnp.maximum(a, b) for a, b in zip(ms[::2], ms[1::2])]
        return ms[0]

    def process(row_v, row):
        zero_hist(1)

        def pAB(i, st):
            umax, off = st
            base = i * (_U * _LANES)
            os_ = [base + t * _LANES for t in range(_U)]
            xs = [row_v[pl.ds(o, _LANES)] for o in os_]
            us = [jnp.maximum(plsc.bitcast(x, jnp.int32), zeros_i)
                  for x in xs]
            idxs = [lax.shift_right_logical(u, 24) * _LANES + lane
                    for u in us]
            for t in range(_U):
                plsc.addupdate_scatter(hists[0], [idxs[t]], ones_i)
            ges = [u >= jnp.int32(_SPEC_BITS) for u in us]
            pcs = [plsc.all_reduce_population_count(g) for g in ges]
            pss = [jnp.squeeze(lax.slice(p, (0,), (1,))) for p in pcs]
            offs = [off]
            for t in range(_U):
                offs.append(offs[-1] + pss[t])
            for t in range(_U):
                plsc.store_compressed(
                    cidx_v.at[pl.ds(offs[t], _LANES)],
                    os_[t] + lane, mask=ges[t]
                )
            return (jnp.maximum(umax, tree_max(us)), offs[_U])

        umax, nspec = lax.fori_loop(
            0, _CHUNKS // _U, pAB, (zeros_i, jnp.int32(0))
        )
        um = jnp.max(umax)
        d_sel, acc = scan_bins(
            lax.shift_right_logical(um, 24), jnp.int32(_K), 1
        )
        kk = jnp.int32(_K) - acc

        def pB(i, off):
            base = i * (_U * _LANES)
            os_ = [base + t * _LANES for t in range(_U)]
            xs = [row_v[pl.ds(o, _LANES)] for o in os_]
            us = [jnp.maximum(plsc.bitcast(x, jnp.int32), zeros_i)
                  for x in xs]
            ges = [lax.shift_right_logical(u, 24) >= d_sel for u in us]
            pcs = [plsc.all_reduce_population_count(g) for g in ges]
            pss = [jnp.squeeze(lax.slice(p, (0,), (1,))) for p in pcs]
            offs = [off]
            for t in range(_U):
                offs.append(offs[-1] + pss[t])
            for t in range(_U):
                plsc.store_compressed(
                    cidx_v.at[pl.ds(offs[t], _LANES)],
                    os_[t] + lane, mask=ges[t]
                )
            return offs[_U]

        spec_ok = d_sel >= jnp.int32(_D_EST)
        nfixq = jnp.where(spec_ok, 0, jnp.int32(_CHUNKS // _U))
        ncand_fb = lax.fori_loop(0, nfixq, pB, jnp.int32(0))
        ncand = jnp.where(spec_ok, nspec, ncand_fb)

        cu = 4
        ncq4 = (ncand + cu * _LANES - 1) // (cu * _LANES)
        prefix = d_sel
        for p in range(1, 4):
            shift = 24 - 8 * p
            hs = shift + 8
            zero_hist(2)

            def pc(ci, umax, shift=shift, hs=hs, prefix=prefix,
                   ncand=ncand):
                cb = ci * (cu * _LANES)
                vms = [(cb + t * _LANES + lane) < ncand
                       for t in range(cu)]
                cidxs = [cidx_v[pl.ds(cb + t * _LANES, _LANES)]
                         & (_COLS - 1) for t in range(cu)]
                xgs = [plsc.load_gather(row_v, [cidxs[t]], mask=vms[t])
                       for t in range(cu)]
                us = [jnp.maximum(plsc.bitcast(x, jnp.int32), zeros_i)
                      for x in xgs]
                cands = [
                    vms[t]
                    & (lax.shift_right_logical(us[t], hs) == prefix)
                    for t in range(cu)
                ]
                dgs = [lax.shift_right_logical(u, shift) & 0xFF
                       for u in us]
                for t in range(cu):
                    plsc.addupdate_scatter(
                        hists[t % 2], [dgs[t] * _LANES + lane], ones_i,
                        mask=cands[t],
                    )
                ms = [jnp.where(cands[t], us[t], zeros_i)
                      for t in range(cu)]
                return jnp.maximum(umax, tree_max(ms))

            umax = lax.fori_loop(0, ncq4, pc, zeros_i)
            um = jnp.max(umax)
            d_sel2, acc = scan_bins(
                lax.shift_right_logical(um, shift) & 0xFF, kk, 2
            )
            kk = kk - acc
            prefix = lax.shift_left(prefix, 8) | d_sel2


        def pr(ci, st, prefix=prefix, kk=kk, ncand=ncand, row=row):
            carry, wcnt = st
            cb = ci * (cu * _LANES)
            vms = [(cb + t * _LANES + lane) < ncand for t in range(cu)]
            cidxs = [cidx_v[pl.ds(cb + t * _LANES, _LANES)]
                     & (_COLS - 1) for t in range(cu)]
            xgs = [plsc.load_gather(row_v, [cidxs[t]], mask=vms[t])
                   for t in range(cu)]
            us = [jnp.maximum(plsc.bitcast(x, jnp.int32), zeros_i)
                  for x in xgs]
            gts = [vms[t] & (us[t] > prefix) for t in range(cu)]
            eqs = [vms[t] & (us[t] == prefix) for t in range(cu)]
            eqis = [jnp.where(e, ones_i, zeros_i) for e in eqs]
            css = [plsc.cumsum(e) for e in eqis]
            pce = [plsc.all_reduce_population_count(e) for e in eqs]
            carries = [carry]
            for t in range(cu):
                carries.append(carries[-1] + pce[t])
            keeps = [
                jnp.logical_or(
                    gts[t], eqs[t] & ((css[t] + carries[t]) <= kk)
                )
                for t in range(cu)
            ]
            keepis = [jnp.where(k, ones_i, zeros_i) for k in keeps]
            kcss = [plsc.cumsum(k) for k in keepis]
            pck = [plsc.all_reduce_population_count(k) for k in keeps]
            wcnts = [wcnt]
            for t in range(cu):
                wcnts.append(wcnts[-1] + pck[t])
            for t in range(cu):
                kpos = wcnts[t] + kcss[t] - keepis[t]
                plsc.store_scatter(
                    pidx_v, [kpos], row * _COLS + cidxs[t],
                    mask=keeps[t],
                )
                plsc.store_scatter(
                    pval_v, [kpos],
                    plsc.bitcast(us[t], jnp.float32), mask=keeps[t],
                )
            return (carries[cu], wcnts[cu])

        lax.fori_loop(0, ncq4, pr, (zeros_i, zeros_i))

    cp0.wait()
    cp1 = pltpu.async_copy(x_hbm.at[r0 + 1], row1_v, sem_in1)
    zcopies = []
    for rr in range(_ROWS_PER_TILE):
        for j in range(_COLS // _ZW):
            zcopies.append(pltpu.async_copy(
                zbuf_v,
                outf_hbm.at[pl.ds((r0 + rr) * _COLS + j * _ZW, _ZW)],
                sem_z,
            ))
    process(row0_v, r0)
    for c in zcopies:
        c.wait()
    pltpu.async_copy(pval_v, outf_hbm.at[pidx_v], sem_s).wait()
    cp1.wait()
    process(row1_v, r0 + 1)
    pltpu.async_copy(pval_v, outf_hbm.at[pidx_v], sem_s).wait()


@jax.jit
def _topk_sc(x):
    mesh = plsc.VectorSubcoreMesh(core_axis_name="c", subcore_axis_name="s")
    fn = pl.kernel(
        _tile_body,
        out_type=jax.ShapeDtypeStruct((_ROWS * _COLS,), jnp.float32),
        mesh=mesh,
        compiler_params=pltpu.CompilerParams(needs_layout_passes=False),
        scratch_types=[
            pltpu.VMEM((_COLS,), jnp.float32),
            pltpu.VMEM((_COLS,), jnp.float32),
            pltpu.VMEM((_COLS + 64,), jnp.int32),
            pltpu.VMEM((_ZW,), jnp.float32),
            pltpu.VMEM((_K,), jnp.int32),
            pltpu.VMEM((_K,), jnp.float32),
            pltpu.VMEM((_NBINS * _LANES,), jnp.int32),
            pltpu.VMEM((_NBINS * _LANES,), jnp.int32),
            pltpu.VMEM((_NBINS * _LANES,), jnp.int32),
            pltpu.VMEM((_NBINS * _LANES,), jnp.int32),
            pltpu.SemaphoreType.DMA,
            pltpu.SemaphoreType.DMA,
            pltpu.SemaphoreType.DMA,
            pltpu.SemaphoreType.DMA,
        ],
    )
    return fn(x).reshape(_ROWS, _COLS)


def kernel(x):
    return _topk_sc(x)


# speedup vs baseline: 1.3782x; 1.0386x over previous
import jax
import jax.numpy as jnp
from jax import lax
from jax.experimental import pallas as pl
from jax.experimental.pallas import tpu as pltpu
from jax.experimental.pallas import tpu_sc as plsc

_ROWS, _COLS = 64, 32768
_K = 64
_LANES = 16
_CHUNKS = _COLS // _LANES
_NBINS = 256
_ROWS_PER_TILE = 2
_U = 16
_D_EST = 0x40
_SPEC_BITS = _D_EST << 24
_ZW = 8192


def _tile_body(x_hbm, outf_hbm, row0_v, row1_v, cidx_v, zbuf_v,
               pidx_v, pval_v, h0, h1, h2, h3,
               sem_in0, sem_in1, sem_z, sem_s):
    hists = (h0, h1, h2, h3)
    cid = lax.axis_index("c")
    sid = lax.axis_index("s")
    wid = sid * 2 + cid
    r0 = wid * _ROWS_PER_TILE

    lane = lax.iota(jnp.int32, _LANES)
    ones_i = jnp.ones((_LANES,), jnp.int32)
    zeros_i = jnp.zeros((_LANES,), jnp.int32)
    zeros_f = jnp.zeros((_LANES,), jnp.float32)

    cp0 = pltpu.async_copy(x_hbm.at[r0], row0_v, sem_in0)

    def zb(j, c):
        base = j * (_U * _LANES)
        for t in range(_U):
            zbuf_v[pl.ds(base + t * _LANES, _LANES)] = zeros_f
        return c

    lax.fori_loop(0, _ZW // (_U * _LANES), zb, jnp.int32(0))

    def bin_total(d, nh):
        s = hists[0][pl.ds(d * _LANES, _LANES)]
        for h in hists[1:nh]:
            s = s + h[pl.ds(d * _LANES, _LANES)]
        return jnp.sum(s)

    def scan_bins(d0, kk, nh):
        def cond(st):
            d, acc = st
            return acc + bin_total(d, nh) < kk

        def body(st):
            d, acc = st
            return d - 1, acc + bin_total(d, nh)

        return lax.while_loop(cond, body, (d0, jnp.int32(0)))

    def zero_hist(nh):
        def zh(j, c):
            base = j * (_U * _LANES)
            for t in range(_U):
                for h in hists[:nh]:
                    h[pl.ds(base + t * _LANES, _LANES)] = zeros_i
            return c

        lax.fori_loop(0, _NBINS // _U, zh, jnp.int32(0))

    def tree_max(ms):
        while len(ms) > 1:
            ms = [jnp.maximum(a, b) for a, b in zip(ms[::2], ms[1::2])]
        return ms[0]

    def process(row_v, row):
        zero_hist(1)

        def pAB(i, st):
            umax, off = st
            base = i * (_U * _LANES)
            os_ = [base + t * _LANES for t in range(_U)]
            xs = [row_v[pl.ds(o, _LANES)] for o in os_]
            us = [jnp.maximum(plsc.bitcast(x, jnp.int32), zeros_i)
                  for x in xs]
            idxs = [lax.shift_right_logical(u, 24) * _LANES + lane
                    for u in us]
            for t in range(_U):
                plsc.addupdate_scatter(hists[0], [idxs[t]], ones_i)
            ges = [u >= jnp.int32(_SPEC_BITS) for u in us]
            pcs = [plsc.all_reduce_population_count(g) for g in ges]
            pss = [jnp.squeeze(lax.slice(p, (0,), (1,))) for p in pcs]
            offs = [off]
            for t in range(_U):
                offs.append(offs[-1] + pss[t])
            for t in range(_U):
                plsc.store_compressed(
                    cidx_v.at[pl.ds(offs[t], _LANES)],
                    os_[t] + lane, mask=ges[t]
                )
            return (jnp.maximum(umax, tree_max(us)), offs[_U])

        umax, nspec = lax.fori_loop(
            0, _CHUNKS // _U, pAB, (zeros_i, jnp.int32(0))
        )
        um = jnp.max(umax)
        d_sel, acc = scan_bins(
            lax.shift_right_logical(um, 24), jnp.int32(_K), 1
        )
        kk = jnp.int32(_K) - acc

        def pB(i, off):
            base = i * (_U * _LANES)
            os_ = [base + t * _LANES for t in range(_U)]
            xs = [row_v[pl.ds(o, _LANES)] for o in os_]
            us = [jnp.maximum(plsc.bitcast(x, jnp.int32), zeros_i)
                  for x in xs]
            ges = [lax.shift_right_logical(u, 24) >= d_sel for u in us]
            pcs = [plsc.all_reduce_population_count(g) for g in ges]
            pss = [jnp.squeeze(lax.slice(p, (0,), (1,))) for p in pcs]
            offs = [off]
            for t in range(_U):
                offs.append(offs[-1] + pss[t])
            for t in range(_U):
                plsc.store_compressed(
                    cidx_v.at[pl.ds(offs[t], _LANES)],
                    os_[t] + lane, mask=ges[t]
                )
            return offs[_U]

        spec_ok = d_sel >= jnp.int32(_D_EST)
        nfixq = jnp.where(spec_ok, 0, jnp.int32(_CHUNKS // _U))
        ncand_fb = lax.fori_loop(0, nfixq, pB, jnp.int32(0))
        ncand = jnp.where(spec_ok, nspec, ncand_fb)

        cu = 4
        ncq4 = (ncand + cu * _LANES - 1) // (cu * _LANES)
        prefix = d_sel
        for p in range(1, 4):
            shift = 24 - 8 * p
            hs = shift + 8
            zero_hist(1)

            def pc(ci, umax, shift=shift, hs=hs, prefix=prefix,
                   ncand=ncand):
                cb = ci * (cu * _LANES)
                vms = [(cb + t * _LANES + lane) < ncand
                       for t in range(cu)]
                cidxs = [cidx_v[pl.ds(cb + t * _LANES, _LANES)]
                         & (_COLS - 1) for t in range(cu)]
                xgs = [plsc.load_gather(row_v, [cidxs[t]], mask=vms[t])
                       for t in range(cu)]
                us = [jnp.maximum(plsc.bitcast(x, jnp.int32), zeros_i)
                      for x in xgs]
                cands = [
                    vms[t]
                    & (lax.shift_right_logical(us[t], hs) == prefix)
                    for t in range(cu)
                ]
                dgs = [lax.shift_right_logical(u, shift) & 0xFF
                       for u in us]
                for t in range(cu):
                    plsc.addupdate_scatter(
                        hists[0], [dgs[t] * _LANES + lane], ones_i,
                        mask=cands[t],
                    )
                ms = [jnp.where(cands[t], us[t], zeros_i)
                      for t in range(cu)]
                return jnp.maximum(umax, tree_max(ms))

            umax = lax.fori_loop(0, ncq4, pc, zeros_i)
            um = jnp.max(umax)
            d_sel2, acc = scan_bins(
                lax.shift_right_logical(um, shift) & 0xFF, kk, 1
            )
            kk = kk - acc
            prefix = lax.shift_left(prefix, 8) | d_sel2


        def pr(ci, st, prefix=prefix, kk=kk, ncand=ncand, row=row):
            carry, wcnt = st
            cb = ci * (cu * _LANES)
            vms = [(cb + t * _LANES + lane) < ncand for t in range(cu)]
            cidxs = [cidx_v[pl.ds(cb + t * _LANES, _LANES)]
                     & (_COLS - 1) for t in range(cu)]
            xgs = [plsc.load_gather(row_v, [cidxs[t]], mask=vms[t])
                   for t in range(cu)]
            us = [jnp.maximum(plsc.bitcast(x, jnp.int32), zeros_i)
                  for x in xgs]
            gts = [vms[t] & (us[t] > prefix) for t in range(cu)]
            eqs = [vms[t] & (us[t] == prefix) for t in range(cu)]
            eqis = [jnp.where(e, ones_i, zeros_i) for e in eqs]
            css = [plsc.cumsum(e) for e in eqis]
            pce = [plsc.all_reduce_population_count(e) for e in eqs]
            carries = [carry]
            for t in range(cu):
                carries.append(carries[-1] + pce[t])
            keeps = [
                jnp.logical_or(
                    gts[t], eqs[t] & ((css[t] + carries[t]) <= kk)
                )
                for t in range(cu)
            ]
            keepis = [jnp.where(k, ones_i, zeros_i) for k in keeps]
            kcss = [plsc.cumsum(k) for k in keepis]
            pck = [plsc.all_reduce_population_count(k) for k in keeps]
            wcnts = [wcnt]
            for t in range(cu):
                wcnts.append(wcnts[-1] + pck[t])
            for t in range(cu):
                kpos = wcnts[t] + kcss[t] - keepis[t]
                plsc.store_scatter(
                    pidx_v, [kpos], row * _COLS + cidxs[t],
                    mask=keeps[t],
                )
                plsc.store_scatter(
                    pval_v, [kpos],
                    plsc.bitcast(us[t], jnp.float32), mask=keeps[t],
                )
            return (carries[cu], wcnts[cu])

        lax.fori_loop(0, ncq4, pr, (zeros_i, zeros_i))

    cp0.wait()
    cp1 = pltpu.async_copy(x_hbm.at[r0 + 1], row1_v, sem_in1)
    zcopies = []
    for rr in range(_ROWS_PER_TILE):
        for j in range(_COLS // _ZW):
            zcopies.append(pltpu.async_copy(
                zbuf_v,
                outf_hbm.at[pl.ds((r0 + rr) * _COLS + j * _ZW, _ZW)],
                sem_z,
            ))
    process(row0_v, r0)
    for c in zcopies:
        c.wait()
    pltpu.async_copy(pval_v, outf_hbm.at[pidx_v], sem_s).wait()
    cp1.wait()
    process(row1_v, r0 + 1)
    pltpu.async_copy(pval_v, outf_hbm.at[pidx_v], sem_s).wait()


@jax.jit
def _topk_sc(x):
    mesh = plsc.VectorSubcoreMesh(core_axis_name="c", subcore_axis_name="s")
    fn = pl.kernel(
        _tile_body,
        out_type=jax.ShapeDtypeStruct((_ROWS * _COLS,), jnp.float32),
        mesh=mesh,
        compiler_params=pltpu.CompilerParams(needs_layout_passes=False),
        scratch_types=[
            pltpu.VMEM((_COLS,), jnp.float32),
            pltpu.VMEM((_COLS,), jnp.float32),
            pltpu.VMEM((_COLS + 64,), jnp.int32),
            pltpu.VMEM((_ZW,), jnp.float32),
            pltpu.VMEM((_K,), jnp.int32),
            pltpu.VMEM((_K,), jnp.float32),
            pltpu.VMEM((_NBINS * _LANES,), jnp.int32),
            pltpu.VMEM((_NBINS * _LANES,), jnp.int32),
            pltpu.VMEM((_NBINS * _LANES,), jnp.int32),
            pltpu.VMEM((_NBINS * _LANES,), jnp.int32),
            pltpu.SemaphoreType.DMA,
            pltpu.SemaphoreType.DMA,
            pltpu.SemaphoreType.DMA,
            pltpu.SemaphoreType.DMA,
        ],
    )
    return fn(x).reshape(_ROWS, _COLS)


def kernel(x):
    return _topk_sc(x)
